# baseline (speedup 1.0000x reference)
import functools

import jax
import jax.numpy as jnp
from jax import lax
from jax.experimental import pallas as pl
from jax.experimental.pallas import tpu as pltpu
from jax.experimental.pallas import tpu_sc as plsc

_NUM_CORES = 2
_NUM_SUBCORES = 16
_NUM_WORKERS = _NUM_CORES * _NUM_SUBCORES

_SREP = 32


def kernel(sequence, table):
    batch, seq = sequence.shape
    max_len, hidden = table.shape
    row = seq * hidden
    b_per_w = batch // _NUM_WORKERS
    n_fill = _SREP // _NUM_SUBCORES
    n_out_dma = b_per_w // _SREP

    tab_flat = table.reshape(-1)

    @functools.partial(
        pl.kernel,
        mesh=plsc.VectorSubcoreMesh(core_axis_name="c", subcore_axis_name="s"),
        out_type=jax.ShapeDtypeStruct((batch, row), jnp.float32),
        scratch_types=[
            pltpu.VMEM((row,), jnp.float32),
            pltpu.VMEM_SHARED((_SREP, row), jnp.float32),
            pltpu.SemaphoreType.DMA,
        ],
    )
    def bcast(tab_hbm, out_hbm, vbuf, shared, sem):
        c = lax.axis_index("c")
        s = lax.axis_index("s")
        wid = s * _NUM_CORES + c
        base = wid * b_per_w
        pltpu.async_copy(tab_hbm.at[pl.ds(0, row)], vbuf, sem).wait()
        fills = [
            pltpu.async_copy(vbuf, shared.at[s * n_fill + r], sem)
            for r in range(n_fill)
        ]
        for f in fills:
            f.wait()
        plsc.subcore_barrier()
        outs = [
            pltpu.async_copy(shared, out_hbm.at[pl.ds(base + j * _SREP, _SREP)], sem)
            for j in range(n_out_dma)
        ]
        for o in outs:
            o.wait()

    out = bcast(tab_flat)
    return out.reshape(batch, seq, hidden)


# speedup vs baseline: 1.0819x; 1.0819x over previous
import functools

import jax
import jax.numpy as jnp
from jax import lax
from jax.experimental import pallas as pl
from jax.experimental.pallas import tpu as pltpu
from jax.experimental.pallas import tpu_sc as plsc

_NUM_CORES = 2
_NUM_SUBCORES = 16
_NUM_WORKERS = _NUM_CORES * _NUM_SUBCORES

_REP = 8


def kernel(sequence, table):
    batch, seq = sequence.shape
    max_len, hidden = table.shape
    row = seq * hidden
    b_per_w = batch // _NUM_WORKERS
    n_out_dma = b_per_w // _REP

    tab_flat = table.reshape(-1)

    @functools.partial(
        pl.kernel,
        mesh=plsc.VectorSubcoreMesh(core_axis_name="c", subcore_axis_name="s"),
        out_type=jax.ShapeDtypeStruct((batch, row), jnp.float32),
        scratch_types=[
            pltpu.VMEM((_REP, row), jnp.float32),
            pltpu.SemaphoreType.DMA,
        ],
    )
    def bcast(tab_hbm, out_hbm, buf, sem):
        wid = lax.axis_index("s") * _NUM_CORES + lax.axis_index("c")
        base = wid * b_per_w
        fills = [
            pltpu.async_copy(tab_hbm.at[pl.ds(0, row)], buf.at[r], sem)
            for r in range(_REP)
        ]
        for f in fills:
            f.wait()
        outs = [
            pltpu.async_copy(buf, out_hbm.at[pl.ds(base + i * _REP, _REP)], sem)
            for i in range(n_out_dma)
        ]
        for o in outs:
            o.wait()

    out = bcast(tab_flat)
    return out.reshape(batch, seq, hidden)
